# baseline (device time: 20004 ns/iter reference)
import jax
import jax.numpy as jnp
from jax import lax
from jax.experimental import pallas as pl
from jax.experimental.pallas import tpu as pltpu

N_DEV = 4
E_LOCAL = 2
N_EXP = 8
N_TOK = 256
D_IN = 128
D_OUT = 256


def kernel(x, router_W, route_idx, expert_W):
    def body(x_ref, rw_ref, idx_ref, ew_ref, out_ref,
             comm_ref, send_sems, recv_sems):
        my_pos = lax.axis_index("i")
        left = lax.rem(my_pos + N_DEV - 1, N_DEV)
        right = lax.rem(my_pos + 1, N_DEV)

        barrier_sem = pltpu.get_barrier_semaphore()
        for nbr in (left, right):
            pl.semaphore_signal(
                barrier_sem, inc=1,
                device_id=(nbr,), device_id_type=pl.DeviceIdType.MESH,
            )
        pl.semaphore_wait(barrier_sem, 2)

        xv = x_ref[:, :]

        scores = jnp.dot(xv, rw_ref[:, :], preferred_element_type=jnp.float32)
        s_max = jnp.max(scores, axis=1, keepdims=True)
        p = jnp.exp(scores - s_max)
        probs = p / jnp.sum(p, axis=1, keepdims=True)

        idx0 = idx_ref[:, 0:1]
        idx1 = idx_ref[:, 1:2]
        iota = lax.broadcasted_iota(jnp.int32, (N_TOK, N_EXP), 1)
        p0 = jnp.sum(jnp.where(iota == idx0, probs, 0.0), axis=1, keepdims=True)
        p1 = jnp.sum(jnp.where(iota == idx1, probs, 0.0), axis=1, keepdims=True)
        gate_sum = p0 + p1

        partial = jnp.zeros((N_TOK, D_OUT), jnp.float32)
        for l in range(E_LOCAL):
            e_g = my_pos * E_LOCAL + l
            sel = (idx0 == e_g) | (idx1 == e_g)
            p_e = jnp.sum(jnp.where(iota == e_g, probs, 0.0),
                          axis=1, keepdims=True)
            w = jnp.where(sel, p_e / gate_sum, 0.0)
            y = jnp.dot(xv, ew_ref[l], preferred_element_type=jnp.float32)
            partial = partial + w * y

        out_ref[:, :] = partial
        comm_ref[0, :, :] = partial

        for h in range(N_DEV - 1):
            rdma = pltpu.make_async_remote_copy(
                src_ref=comm_ref.at[h],
                dst_ref=comm_ref.at[h + 1],
                send_sem=send_sems.at[h],
                recv_sem=recv_sems.at[h],
                device_id=(right,),
                device_id_type=pl.DeviceIdType.MESH,
            )
            rdma.start()
            rdma.wait()
            out_ref[:, :] = out_ref[:, :] + comm_ref[h + 1, :, :]

    return pl.pallas_call(
        body,
        out_shape=jax.ShapeDtypeStruct((N_TOK, D_OUT), jnp.float32),
        in_specs=[pl.BlockSpec(memory_space=pltpu.VMEM)] * 4,
        out_specs=pl.BlockSpec(memory_space=pltpu.VMEM),
        scratch_shapes=[
            pltpu.VMEM((N_DEV, N_TOK, D_OUT), jnp.float32),
            pltpu.SemaphoreType.DMA((N_DEV - 1,)),
            pltpu.SemaphoreType.DMA((N_DEV - 1,)),
        ],
        compiler_params=pltpu.CompilerParams(collective_id=0),
    )(x, router_W, route_idx, expert_W)


# device time: 15086 ns/iter; 1.3260x vs baseline; 1.3260x over previous
import jax
import jax.numpy as jnp
from jax import lax
from jax.experimental import pallas as pl
from jax.experimental.pallas import tpu as pltpu

N_DEV = 4
E_LOCAL = 2
N_EXP = 8
N_TOK = 256
D_IN = 128
D_OUT = 256


def kernel(x, router_W, route_idx, expert_W):
    def body(x_ref, rw_ref, idx_ref, ew_ref, out_ref,
             comm_ref, send_sems, recv_sems):
        my_pos = lax.axis_index("i")
        left = lax.rem(my_pos + N_DEV - 1, N_DEV)
        right = lax.rem(my_pos + 1, N_DEV)

        barrier_sem = pltpu.get_barrier_semaphore()
        for nbr in (left, right):
            pl.semaphore_signal(
                barrier_sem, inc=1,
                device_id=(nbr,), device_id_type=pl.DeviceIdType.MESH,
            )
        pl.semaphore_wait(barrier_sem, 2)

        xv = x_ref[:, :]

        scores = jnp.dot(xv, rw_ref[:, :], preferred_element_type=jnp.float32)
        s_max = jnp.max(scores, axis=1, keepdims=True)
        p = jnp.exp(scores - s_max)
        probs = p / jnp.sum(p, axis=1, keepdims=True)

        idx0 = idx_ref[:, 0:1]
        idx1 = idx_ref[:, 1:2]
        iota = lax.broadcasted_iota(jnp.int32, (N_TOK, N_EXP), 1)
        p0 = jnp.sum(jnp.where(iota == idx0, probs, 0.0), axis=1, keepdims=True)
        p1 = jnp.sum(jnp.where(iota == idx1, probs, 0.0), axis=1, keepdims=True)
        gate_sum = p0 + p1

        partial = jnp.zeros((N_TOK, D_OUT), jnp.float32)
        for l in range(E_LOCAL):
            e_g = my_pos * E_LOCAL + l
            sel = (idx0 == e_g) | (idx1 == e_g)
            p_e = jnp.sum(jnp.where(iota == e_g, probs, 0.0),
                          axis=1, keepdims=True)
            w = jnp.where(sel, p_e / gate_sum, 0.0)
            y = jnp.dot(xv, ew_ref[l], preferred_element_type=jnp.float32)
            partial = partial + w * y

        out_ref[:, :] = partial

        partners = (jnp.bitwise_xor(my_pos, 1), 3 - my_pos)
        for s in range(2):
            rdma = pltpu.make_async_remote_copy(
                src_ref=out_ref,
                dst_ref=comm_ref.at[s],
                send_sem=send_sems.at[s],
                recv_sem=recv_sems.at[s],
                device_id=(partners[s],),
                device_id_type=pl.DeviceIdType.MESH,
            )
            rdma.start()
            rdma.wait()
            out_ref[:, :] = out_ref[:, :] + comm_ref[s, :, :]

    return pl.pallas_call(
        body,
        out_shape=jax.ShapeDtypeStruct((N_TOK, D_OUT), jnp.float32),
        in_specs=[pl.BlockSpec(memory_space=pltpu.VMEM)] * 4,
        out_specs=pl.BlockSpec(memory_space=pltpu.VMEM),
        scratch_shapes=[
            pltpu.VMEM((2, N_TOK, D_OUT), jnp.float32),
            pltpu.SemaphoreType.DMA((2,)),
            pltpu.SemaphoreType.DMA((2,)),
        ],
        compiler_params=pltpu.CompilerParams(collective_id=0),
    )(x, router_W, route_idx, expert_W)


# device time: 13689 ns/iter; 1.4613x vs baseline; 1.1021x over previous
import jax
import jax.numpy as jnp
from jax import lax
from jax.experimental import pallas as pl
from jax.experimental.pallas import tpu as pltpu

N_DEV = 4
E_LOCAL = 2
N_EXP = 8
N_TOK = 256
D_IN = 128
D_OUT = 256


def kernel(x, router_W, route_idx, expert_W):
    def body(x_ref, rw_ref, idx_ref, ew_ref, out_ref,
             comm_ref, send_sems, recv_sems):
        my_pos = lax.axis_index("i")
        left = lax.rem(my_pos + N_DEV - 1, N_DEV)
        right = lax.rem(my_pos + 1, N_DEV)

        barrier_sem = pltpu.get_barrier_semaphore()
        for nbr in (left, right):
            pl.semaphore_signal(
                barrier_sem, inc=1,
                device_id=(nbr,), device_id_type=pl.DeviceIdType.MESH,
            )
        pl.semaphore_wait(barrier_sem, 2)

        xv = x_ref[:, :]

        scores = jnp.dot(xv, rw_ref[:, :], preferred_element_type=jnp.float32)
        s_max = jnp.max(scores, axis=1, keepdims=True)
        p = jnp.exp(scores - s_max)
        probs = p / jnp.sum(p, axis=1, keepdims=True)

        idx0 = idx_ref[:, 0:1]
        idx1 = idx_ref[:, 1:2]
        iota = lax.broadcasted_iota(jnp.int32, (N_TOK, N_EXP), 1)
        p0 = jnp.sum(jnp.where(iota == idx0, probs, 0.0), axis=1, keepdims=True)
        p1 = jnp.sum(jnp.where(iota == idx1, probs, 0.0), axis=1, keepdims=True)
        gate_sum = p0 + p1

        partial = jnp.zeros((N_TOK, D_OUT), jnp.float32)
        for l in range(E_LOCAL):
            e_g = my_pos * E_LOCAL + l
            sel = (idx0 == e_g) | (idx1 == e_g)
            p_e = jnp.sum(jnp.where(iota == e_g, probs, 0.0),
                          axis=1, keepdims=True)
            w = jnp.where(sel, p_e / gate_sum, 0.0)
            y = jnp.dot(xv, ew_ref[l], preferred_element_type=jnp.float32)
            partial = partial + w * y

        out_ref[:, :] = partial

        partners = (jnp.bitwise_xor(my_pos, 1), 3 - my_pos)
        NC = 2
        ROWS = N_TOK // NC

        def chunk_rdma(s, c):
            return pltpu.make_async_remote_copy(
                src_ref=out_ref.at[pl.ds(c * ROWS, ROWS), :],
                dst_ref=comm_ref.at[s, c],
                send_sem=send_sems.at[s, c],
                recv_sem=recv_sems.at[s, c],
                device_id=(partners[s],),
                device_id_type=pl.DeviceIdType.MESH,
            )

        step1 = [chunk_rdma(0, c) for c in range(NC)]
        for c in range(NC):
            step1[c].start()
        step2 = []
        for c in range(NC):
            step1[c].wait()
            out_ref[pl.ds(c * ROWS, ROWS), :] += comm_ref[0, c]
            r2 = chunk_rdma(1, c)
            r2.start()
            step2.append(r2)
        for c in range(NC):
            step2[c].wait()
            out_ref[pl.ds(c * ROWS, ROWS), :] += comm_ref[1, c]

    return pl.pallas_call(
        body,
        out_shape=jax.ShapeDtypeStruct((N_TOK, D_OUT), jnp.float32),
        in_specs=[pl.BlockSpec(memory_space=pltpu.VMEM)] * 4,
        out_specs=pl.BlockSpec(memory_space=pltpu.VMEM),
        scratch_shapes=[
            pltpu.VMEM((2, 2, N_TOK // 2, D_OUT), jnp.float32),
            pltpu.SemaphoreType.DMA((2, 2)),
            pltpu.SemaphoreType.DMA((2, 2)),
        ],
        compiler_params=pltpu.CompilerParams(collective_id=0),
    )(x, router_W, route_idx, expert_W)


# device time: 11212 ns/iter; 1.7842x vs baseline; 1.2209x over previous
import jax
import jax.numpy as jnp
from jax import lax
from jax.experimental import pallas as pl
from jax.experimental.pallas import tpu as pltpu

N_DEV = 4
E_LOCAL = 2
N_EXP = 8
N_TOK = 256
D_IN = 128
D_OUT = 256
NC = 2
ROWS = N_TOK // NC


def kernel(x, router_W, route_idx, expert_W):
    def body(x_ref, rw_ref, idx_ref, ew_ref, out_ref,
             comm_ref, sbuf_ref, send_sems, recv_sems):
        my_pos = lax.axis_index("i")
        partners = (jnp.bitwise_xor(my_pos, 1), 3 - my_pos)

        barrier_sem = pltpu.get_barrier_semaphore()
        for s in range(2):
            pl.semaphore_signal(
                barrier_sem, inc=1,
                device_id=(partners[s],), device_id_type=pl.DeviceIdType.MESH,
            )

        xv = x_ref[:, :]

        scores = jnp.dot(xv, rw_ref[:, :], preferred_element_type=jnp.float32)
        s_max = jnp.max(scores, axis=1, keepdims=True)
        p = jnp.exp(scores - s_max)
        probs = p / jnp.sum(p, axis=1, keepdims=True)

        idx0 = idx_ref[:, 0:1]
        idx1 = idx_ref[:, 1:2]
        iota = lax.broadcasted_iota(jnp.int32, (N_TOK, N_EXP), 1)
        p0 = jnp.sum(jnp.where(iota == idx0, probs, 0.0), axis=1, keepdims=True)
        p1 = jnp.sum(jnp.where(iota == idx1, probs, 0.0), axis=1, keepdims=True)
        gate_sum = p0 + p1

        gates = []
        for l in range(E_LOCAL):
            e_g = my_pos * E_LOCAL + l
            sel = (idx0 == e_g) | (idx1 == e_g)
            p_e = jnp.sum(jnp.where(iota == e_g, probs, 0.0),
                          axis=1, keepdims=True)
            gates.append(jnp.where(sel, p_e / gate_sum, 0.0))

        def chunk_rdma(s, c):
            return pltpu.make_async_remote_copy(
                src_ref=sbuf_ref.at[s, c],
                dst_ref=comm_ref.at[s, c],
                send_sem=send_sems.at[s, c],
                recv_sem=recv_sems.at[s, c],
                device_id=(partners[s],),
                device_id_type=pl.DeviceIdType.MESH,
            )

        step1 = []
        for c in range(NC):
            r = slice(c * ROWS, (c + 1) * ROWS)
            x_c = xv[r]
            part = (gates[0][r] * jnp.dot(x_c, ew_ref[0],
                                          preferred_element_type=jnp.float32)
                    + gates[1][r] * jnp.dot(x_c, ew_ref[1],
                                            preferred_element_type=jnp.float32))
            out_ref[r, :] = part
            sbuf_ref[0, c] = part.astype(jnp.bfloat16)
            if c == 0:
                pl.semaphore_wait(barrier_sem, 2)
            rdma = chunk_rdma(0, c)
            rdma.start()
            step1.append(rdma)

        step2 = []
        for c in range(NC):
            r = slice(c * ROWS, (c + 1) * ROWS)
            step1[c].wait()
            acc = out_ref[r, :] + comm_ref[0, c].astype(jnp.float32)
            out_ref[r, :] = acc
            sbuf_ref[1, c] = acc.astype(jnp.bfloat16)
            rdma = chunk_rdma(1, c)
            rdma.start()
            step2.append(rdma)

        for c in range(NC):
            r = slice(c * ROWS, (c + 1) * ROWS)
            step2[c].wait()
            out_ref[r, :] += comm_ref[1, c].astype(jnp.float32)

    return pl.pallas_call(
        body,
        out_shape=jax.ShapeDtypeStruct((N_TOK, D_OUT), jnp.float32),
        in_specs=[pl.BlockSpec(memory_space=pltpu.VMEM)] * 4,
        out_specs=pl.BlockSpec(memory_space=pltpu.VMEM),
        scratch_shapes=[
            pltpu.VMEM((2, NC, ROWS, D_OUT), jnp.bfloat16),
            pltpu.VMEM((2, NC, ROWS, D_OUT), jnp.bfloat16),
            pltpu.SemaphoreType.DMA((2, NC)),
            pltpu.SemaphoreType.DMA((2, NC)),
        ],
        compiler_params=pltpu.CompilerParams(collective_id=0),
    )(x, router_W, route_idx, expert_W)


# device time: 10680 ns/iter; 1.8730x vs baseline; 1.0498x over previous
import jax
import jax.numpy as jnp
from jax import lax
from jax.experimental import pallas as pl
from jax.experimental.pallas import tpu as pltpu

N_DEV = 4
E_LOCAL = 2
N_EXP = 8
N_TOK = 256
D_IN = 128
D_OUT = 256
NC = 2
ROWS = N_TOK // NC


def kernel(x, router_W, route_idx, expert_W):
    def body(x_ref, rw_ref, idx_ref, ew_ref, out_ref,
             comm_ref, sbuf_ref, send_sems, recv_sems):
        my_pos = lax.axis_index("i")
        partners = (jnp.bitwise_xor(my_pos, 1), 3 - my_pos)

        barrier_sem = pltpu.get_barrier_semaphore()
        for s in range(2):
            pl.semaphore_signal(
                barrier_sem, inc=1,
                device_id=(partners[s],), device_id_type=pl.DeviceIdType.MESH,
            )

        xv = x_ref[:, :]

        scores = jnp.dot(xv, rw_ref[:, :], preferred_element_type=jnp.float32)
        s_max = jnp.max(scores, axis=1, keepdims=True)
        p = jnp.exp(scores - s_max)
        probs = p / jnp.sum(p, axis=1, keepdims=True)

        idx0 = idx_ref[:, 0:1]
        idx1 = idx_ref[:, 1:2]
        iota = lax.broadcasted_iota(jnp.int32, (N_TOK, N_EXP), 1)
        p0 = jnp.sum(jnp.where(iota == idx0, probs, 0.0), axis=1, keepdims=True)
        p1 = jnp.sum(jnp.where(iota == idx1, probs, 0.0), axis=1, keepdims=True)
        gate_sum = p0 + p1

        gates = []
        for l in range(E_LOCAL):
            e_g = my_pos * E_LOCAL + l
            sel = (idx0 == e_g) | (idx1 == e_g)
            p_e = jnp.sum(jnp.where(iota == e_g, probs, 0.0),
                          axis=1, keepdims=True)
            gates.append(jnp.where(sel, p_e / gate_sum, 0.0))

        def chunk_rdma(s, c):
            return pltpu.make_async_remote_copy(
                src_ref=sbuf_ref.at[s, c],
                dst_ref=comm_ref.at[s, c],
                send_sem=send_sems.at[s, c],
                recv_sem=recv_sems.at[s, c],
                device_id=(partners[(s + c) % 2],),
                device_id_type=pl.DeviceIdType.MESH,
            )

        x_b = xv.astype(jnp.bfloat16)
        ew_b = [ew_ref[l].astype(jnp.bfloat16) for l in range(E_LOCAL)]
        step1 = []
        for c in range(NC):
            r = slice(c * ROWS, (c + 1) * ROWS)
            x_c = x_b[r]
            part = (gates[0][r] * jnp.dot(x_c, ew_b[0],
                                          preferred_element_type=jnp.float32)
                    + gates[1][r] * jnp.dot(x_c, ew_b[1],
                                            preferred_element_type=jnp.float32))
            out_ref[r, :] = part
            sbuf_ref[0, c] = part.astype(jnp.bfloat16)
            if c == 0:
                pl.semaphore_wait(barrier_sem, 2)
            rdma = chunk_rdma(0, c)
            rdma.start()
            step1.append(rdma)

        step2 = []
        for c in range(NC):
            r = slice(c * ROWS, (c + 1) * ROWS)
            step1[c].wait()
            acc = out_ref[r, :] + comm_ref[0, c].astype(jnp.float32)
            out_ref[r, :] = acc
            sbuf_ref[1, c] = acc.astype(jnp.bfloat16)
            rdma = chunk_rdma(1, c)
            rdma.start()
            step2.append(rdma)

        for c in range(NC):
            r = slice(c * ROWS, (c + 1) * ROWS)
            step2[c].wait()
            out_ref[r, :] += comm_ref[1, c].astype(jnp.float32)

    return pl.pallas_call(
        body,
        out_shape=jax.ShapeDtypeStruct((N_TOK, D_OUT), jnp.float32),
        in_specs=[pl.BlockSpec(memory_space=pltpu.VMEM)] * 4,
        out_specs=pl.BlockSpec(memory_space=pltpu.VMEM),
        scratch_shapes=[
            pltpu.VMEM((2, NC, ROWS, D_OUT), jnp.bfloat16),
            pltpu.VMEM((2, NC, ROWS, D_OUT), jnp.bfloat16),
            pltpu.SemaphoreType.DMA((2, NC)),
            pltpu.SemaphoreType.DMA((2, NC)),
        ],
        compiler_params=pltpu.CompilerParams(collective_id=0),
    )(x, router_W, route_idx, expert_W)


# device time: 10412 ns/iter; 1.9212x vs baseline; 1.0257x over previous
import jax
import jax.numpy as jnp
from jax import lax
from jax.experimental import pallas as pl
from jax.experimental.pallas import tpu as pltpu

N_DEV = 4
E_LOCAL = 2
N_EXP = 8
N_TOK = 256
D_IN = 128
D_OUT = 256
NC = 4
ROWS = N_TOK // NC


def kernel(x, router_W, route_idx, expert_W):
    def body(x_ref, rw_ref, idx_ref, ew_ref, out_ref,
             comm_ref, sbuf_ref, send_sems, recv_sems):
        my_pos = lax.axis_index("i")
        partners = (jnp.bitwise_xor(my_pos, 1), 3 - my_pos)

        barrier_sem = pltpu.get_barrier_semaphore()
        for s in range(2):
            pl.semaphore_signal(
                barrier_sem, inc=1,
                device_id=(partners[s],), device_id_type=pl.DeviceIdType.MESH,
            )

        xv = x_ref[:, :]

        scores = jnp.dot(xv, rw_ref[:, :], preferred_element_type=jnp.float32)
        s_max = jnp.max(scores, axis=1, keepdims=True)
        p = jnp.exp(scores - s_max)
        probs = p / jnp.sum(p, axis=1, keepdims=True)

        idx0 = idx_ref[:, 0:1]
        idx1 = idx_ref[:, 1:2]
        iota = lax.broadcasted_iota(jnp.int32, (N_TOK, N_EXP), 1)
        p0 = jnp.sum(jnp.where(iota == idx0, probs, 0.0), axis=1, keepdims=True)
        p1 = jnp.sum(jnp.where(iota == idx1, probs, 0.0), axis=1, keepdims=True)
        gate_sum = p0 + p1

        gates = []
        for l in range(E_LOCAL):
            e_g = my_pos * E_LOCAL + l
            sel = (idx0 == e_g) | (idx1 == e_g)
            p_e = jnp.sum(jnp.where(iota == e_g, probs, 0.0),
                          axis=1, keepdims=True)
            gates.append(jnp.where(sel, p_e / gate_sum, 0.0))

        def chunk_rdma(s, c):
            return pltpu.make_async_remote_copy(
                src_ref=sbuf_ref.at[s, c],
                dst_ref=comm_ref.at[s, c],
                send_sem=send_sems.at[s, c],
                recv_sem=recv_sems.at[s, c],
                device_id=(partners[(s + c) % 2],),
                device_id_type=pl.DeviceIdType.MESH,
            )

        x_b = xv.astype(jnp.bfloat16)
        ew_b = [ew_ref[l].astype(jnp.bfloat16) for l in range(E_LOCAL)]
        step1 = []
        for c in range(NC):
            r = slice(c * ROWS, (c + 1) * ROWS)
            x_c = x_b[r]
            part = (gates[0][r] * jnp.dot(x_c, ew_b[0],
                                          preferred_element_type=jnp.float32)
                    + gates[1][r] * jnp.dot(x_c, ew_b[1],
                                            preferred_element_type=jnp.float32))
            out_ref[r, :] = part
            sbuf_ref[0, c] = part.astype(jnp.bfloat16)
            if c == 0:
                pl.semaphore_wait(barrier_sem, 2)
            rdma = chunk_rdma(0, c)
            rdma.start()
            step1.append(rdma)

        step2 = []
        for c in range(NC):
            r = slice(c * ROWS, (c + 1) * ROWS)
            step1[c].wait()
            acc = out_ref[r, :] + comm_ref[0, c].astype(jnp.float32)
            out_ref[r, :] = acc
            sbuf_ref[1, c] = acc.astype(jnp.bfloat16)
            rdma = chunk_rdma(1, c)
            rdma.start()
            step2.append(rdma)

        for c in range(NC):
            r = slice(c * ROWS, (c + 1) * ROWS)
            step2[c].wait()
            out_ref[r, :] += comm_ref[1, c].astype(jnp.float32)

    return pl.pallas_call(
        body,
        out_shape=jax.ShapeDtypeStruct((N_TOK, D_OUT), jnp.float32),
        in_specs=[pl.BlockSpec(memory_space=pltpu.VMEM)] * 4,
        out_specs=pl.BlockSpec(memory_space=pltpu.VMEM),
        scratch_shapes=[
            pltpu.VMEM((2, NC, ROWS, D_OUT), jnp.bfloat16),
            pltpu.VMEM((2, NC, ROWS, D_OUT), jnp.bfloat16),
            pltpu.SemaphoreType.DMA((2, NC)),
            pltpu.SemaphoreType.DMA((2, NC)),
        ],
        compiler_params=pltpu.CompilerParams(collective_id=0),
    )(x, router_W, route_idx, expert_W)
